# baseline (speedup 1.0000x reference)
import functools

import jax
import jax.numpy as jnp
from jax import lax
from jax.experimental import pallas as pl
from jax.experimental.pallas import tpu as pltpu
from jax.experimental.pallas import tpu_sc as plsc

U = 50000
NN = 100000
D = 32
E = 1600000
B = 4096
LMBD_C = 1e-4
HALF = 50000
ACC_ROWS = 50176
E_PAD = 1638400
P_C = 28672
SUP = 2048
NSUP = 2 * P_C // SUP
CPS = SUP // 128
NBUF = 4
NSC = 2
NGRP = CPS // NBUF
NC = 2
NS = 16
_mesh = plsc.VectorSubcoreMesh(
    core_axis_name="c", subcore_axis_name="s", num_cores=NC, num_subcores=NS)


def _layer_body(src_hbm, dst_hbm, ew_hbm, tab_hbm, out_hbm,
                st_src0, st_src1, st_dst0, st_dst1, st_ew0, st_ew1,
                sidx0, sidx1, sidx2, sidx3, didx0, didx1, didx2, didx3,
                rows0, rows1, rows2, rows3, wrows0, wrows1,
                acc, isem, gsem, ssem):
    st_src = (st_src0, st_src1)
    st_dst = (st_dst0, st_dst1)
    st_ew = (st_ew0, st_ew1)
    sidx = (sidx0, sidx1, sidx2, sidx3)
    didx = (didx0, didx1, didx2, didx3)
    rows = (rows0, rows1, rows2, rows3)
    wrows = (wrows0, wrows1)

    c = lax.axis_index("c")
    s = lax.axis_index("s")
    lane = lax.iota(jnp.int32, 16)
    zv = jnp.zeros((16,), jnp.float32)

    def _zb(i, carry):
        wrows0[i, 0:16] = zv
        wrows0[i, 16:32] = zv
        return carry
    lax.fori_loop(0, 128, _zb, 0)

    def _za(i, carry):
        off = pl.multiple_of(s * 3072 + i * 128, 128)
        pltpu.sync_copy(wrows0, acc.at[pl.ds(off, 128)])
        return carry
    lax.fori_loop(0, 24, _za, 0)

    @pl.when(s < 8)
    def _zrem():
        off = pl.multiple_of(49152 + s * 128, 128)
        pltpu.sync_copy(wrows0, acc.at[pl.ds(off, 128)])
    plsc.subcore_barrier()

    lo = c * HALF
    rbase = (c * 32 + 2 * s) * (P_C // 128)

    def _stage_start(u, par):
        row_base = pl.multiple_of(rbase + u * CPS, 8)
        pltpu.async_copy(src_hbm.at[pl.ds(row_base, CPS)], st_src[par], isem)
        pltpu.async_copy(dst_hbm.at[pl.ds(row_base, CPS)], st_dst[par], isem)
        pltpu.async_copy(ew_hbm.at[pl.ds(row_base, CPS)], st_ew[par], isem)

    def _stage_wait(u, par):
        row_base = pl.multiple_of(rbase + u * CPS, 8)
        pltpu.make_async_copy(src_hbm.at[pl.ds(row_base, CPS)], st_src[par], isem).wait()
        pltpu.make_async_copy(dst_hbm.at[pl.ds(row_base, CPS)], st_dst[par], isem).wait()
        pltpu.make_async_copy(ew_hbm.at[pl.ds(row_base, CPS)], st_ew[par], isem).wait()

    def _run_super(u, par):
        _stage_wait(u, par)

        def _grp(g, carry):
            gg = u * NGRP + g

            @pl.when(gg > 0)
            def _wait_prev():
                pltpu.make_async_copy(wrows[0], acc.at[didx[2]], ssem.at[0]).wait()
                pltpu.make_async_copy(wrows[1], acc.at[didx[3]], ssem.at[1]).wait()

            gdesc = []
            for b in range(NBUF):
                k = g * NBUF + b
                for j in range(8):
                    slj = pl.ds(j * 16, 16)
                    sidx[b][slj] = st_src[par][k, slj]
                    dv = st_dst[par][k, slj] - lo
                    m = (dv >= 0) & (dv < HALF)
                    didx[b][slj] = jnp.where(m, dv, HALF + lane + j * 16)
                gdesc.append(pltpu.async_copy(tab_hbm.at[sidx[b]], rows[b], gsem.at[b]))
            for b in range(NBUF):
                sb = b & 1
                if b >= NSC:
                    pltpu.make_async_copy(wrows[sb], acc.at[didx[b - NSC]], ssem.at[sb]).wait()
                gdesc[b].wait()
                k = g * NBUF + b
                for j in range(8):
                    w16 = st_ew[par][k, pl.ds(j * 16, 16)]
                    for i in range(16):
                        r = j * 16 + i
                        ws = w16[i]
                        row = rows[b][r, :].astype(jnp.float32)
                        wrows[sb][r, 0:16] = row[0:16] * ws
                        wrows[sb][r, 16:32] = row[16:32] * ws
                pltpu.async_copy(wrows[sb], acc.at[didx[b]], ssem.at[sb], add=True)
            return carry
        lax.fori_loop(0, NGRP, _grp, 0)

    _stage_start(0, 0)

    def _pair(m, carry):
        for par in range(2):
            u = 2 * m + par

            @pl.when(u + 1 < NSUP)
            def _prefetch():
                _stage_start(u + 1, 1 - par)
            _run_super(u, par)
        return carry
    lax.fori_loop(0, NSUP // 2, _pair, 0)

    pltpu.make_async_copy(wrows[0], acc.at[didx[2]], ssem.at[0]).wait()
    pltpu.make_async_copy(wrows[1], acc.at[didx[3]], ssem.at[1]).wait()
    plsc.subcore_barrier()

    def _pack_store(off, n):
        pltpu.sync_copy(acc.at[pl.ds(off, n)], wrows0.at[pl.ds(0, n)])

        def _pk(r, carry):
            rows0[r, :] = wrows0[r, :].astype(jnp.bfloat16)
            return carry
        lax.fori_loop(0, n, _pk, 0)
        pltpu.sync_copy(rows0.at[pl.ds(0, n)],
                        out_hbm.at[pl.ds(pl.multiple_of(lo + off, 8), n)])

    nfull = jnp.where(s == 15, 15, 25)

    def _co(i, carry):
        _pack_store(pl.multiple_of(s * 3200 + i * 128, 128), 128)
        return carry
    lax.fori_loop(0, nfull, _co, 0)

    @pl.when(s == 15)
    def _tail():
        _pack_store(15 * 3200 + 15 * 128, 80)


_layer = functools.partial(
    pl.kernel,
    out_type=jax.ShapeDtypeStruct((NN, D), jnp.bfloat16),
    mesh=_mesh,
    scratch_types=(
        [pltpu.VMEM((CPS, 128), jnp.int32)] * 4
        + [pltpu.VMEM((CPS, 128), jnp.float32)] * 2
        + [pltpu.VMEM((128,), jnp.int32)] * 8
        + [pltpu.VMEM((128, D), jnp.bfloat16)] * 4
        + [pltpu.VMEM((128, D), jnp.float32)] * 2
        + [
            pltpu.VMEM_SHARED((ACC_ROWS, D), jnp.float32),
            pltpu.SemaphoreType.DMA,
            pltpu.SemaphoreType.DMA((NBUF,)),
            pltpu.SemaphoreType.DMA((NSC,)),
        ]
    ),
    compiler_params=pltpu.CompilerParams(use_tc_tiling_on_sc=False),
)(_layer_body)


def _part_body(src_hbm, dst_hbm, ew_hbm, osrc, odst, oew,
               st_src, st_dst, st_ew, cs0, cs1, cd0, cd1, cw0, cw1):
    c = lax.axis_index("c")
    s = lax.axis_index("s")
    wid = s * NC + c
    zi = jnp.zeros((16,), jnp.int32)
    zf = jnp.zeros((16,), jnp.float32)
    cb = ((cs0, cd0, cw0), (cs1, cd1, cw1))

    def _flush(h, w):
        bufs = cb[h]
        off = pl.multiple_of(
            (h * 32 + wid) * P_C + jnp.minimum(w, P_C - 2048), 8)
        pltpu.sync_copy(bufs[0].at[pl.ds(0, 2048)], osrc.at[pl.ds(off, 2048)])
        pltpu.sync_copy(bufs[1].at[pl.ds(0, 2048)], odst.at[pl.ds(off, 2048)])
        pltpu.sync_copy(bufs[2].at[pl.ds(0, 2048)], oew.at[pl.ds(off, 2048)])

    def _super(u, carry):
        off = pl.multiple_of(wid * 51200 + u * 2048, 8)
        pltpu.sync_copy(src_hbm.at[pl.ds(off, 2048)], st_src)
        pltpu.sync_copy(dst_hbm.at[pl.ds(off, 2048)], st_dst)
        pltpu.sync_copy(ew_hbm.at[pl.ds(off, 2048)], st_ew)

        def _gr(g, car2):
            cnt0, cnt1, w0, w1 = car2
            sl = pl.ds(g * 16, 16)
            sv = st_src[sl]
            dv = st_dst[sl]
            wv = st_ew[sl]
            nz = wv != 0.0
            m0 = (dv < HALF) & nz
            m1 = (dv >= HALF) & nz
            cnts = [cnt0, cnt1]
            for h, mh in ((0, m0), (1, m1)):
                bufs = cb[h]
                cnt = cnts[h]
                plsc.store_compressed(bufs[0].at[pl.ds(cnt, 16)], sv, mask=mh)
                plsc.store_compressed(bufs[1].at[pl.ds(cnt, 16)], dv, mask=mh)
                plsc.store_compressed(bufs[2].at[pl.ds(cnt, 16)], wv, mask=mh)
                cnts[h] = cnt + plsc.all_reduce_population_count(mh)[0]
            cnt0, cnt1 = cnts
            ws = [w0, w1]
            for h in (0, 1):
                cnt = cnts[h]
                full = cnt >= 2048
                w = ws[h]

                @pl.when(full)
                def _do_flush():
                    _flush(h, w)
                    bufs = cb[h]
                    for bi in range(3):
                        bufs[bi][pl.ds(0, 16)] = bufs[bi][pl.ds(2048, 16)]
                cnts[h] = jnp.where(full, cnt - 2048, cnt)
                ws[h] = jnp.where(full, w + 2048, w)
            return (cnts[0], cnts[1], ws[0], ws[1])
        return lax.fori_loop(0, 128, _gr, carry)

    z32 = jnp.int32(0)
    cnt0, cnt1, w0, w1 = lax.fori_loop(0, 25, _super, (z32, z32, z32, z32))

    for h, cnt, w in ((0, cnt0, w0), (1, cnt1, w1)):
        bufs = cb[h]

        def _zt(i, carry):
            start = cnt + i * 16
            bufs[0][pl.ds(start, 16)] = zi
            bufs[1][pl.ds(start, 16)] = zi
            bufs[2][pl.ds(start, 16)] = zf
            return carry
        lax.fori_loop(0, (2048 - cnt + 15) // 16, _zt, 0)
        _flush(h, w)
        w = w + 2048

        def _zb(i, carry):
            sl = pl.ds(i * 16, 16)
            bufs[0][sl] = zi
            bufs[1][sl] = zi
            bufs[2][sl] = zf
            return carry
        lax.fori_loop(0, 128, _zb, 0)
        ndz = jnp.maximum(0, (P_C - w) // 2048)

        def _zfl(i, carry):
            _flush(h, jnp.minimum(w + i * 2048, P_C - 2048))
            return carry
        lax.fori_loop(0, ndz, _zfl, 0)


_part = functools.partial(
    pl.kernel,
    out_type=(jax.ShapeDtypeStruct((2 * 32 * P_C,), jnp.int32),
              jax.ShapeDtypeStruct((2 * 32 * P_C,), jnp.int32),
              jax.ShapeDtypeStruct((2 * 32 * P_C,), jnp.float32)),
    mesh=_mesh,
    scratch_types=(
        [pltpu.VMEM((2048,), jnp.int32)] * 2
        + [pltpu.VMEM((2048,), jnp.float32)]
        + [pltpu.VMEM((2064,), jnp.int32)] * 4
        + [pltpu.VMEM((2064,), jnp.float32)] * 2
    ),
    compiler_params=pltpu.CompilerParams(
        use_tc_tiling_on_sc=False, needs_layout_passes=False),
)(_part_body)


def _score_body(e0, e1, e2, e3, uid, iid, nid, m_out, reg_out,
                idx_v, ue, pe, ne, tmpb, regv, sem):
    c = lax.axis_index("c")
    s = lax.axis_index("s")
    wid = s * NC + c
    b0 = pl.multiple_of(wid * 128, 128)

    def load_set(ids_hbm, buf):
        pltpu.sync_copy(ids_hbm.at[pl.ds(b0, 128)], idx_v)
        racc = jnp.zeros((16,), jnp.float32)
        for ti, t in enumerate((e0, e1, e2, e3)):
            pltpu.async_copy(t.at[idx_v], tmpb, sem).wait()
            if ti == 0:
                def _row0(r, acc2):
                    row = tmpb[r, :].astype(jnp.float32)
                    va = row[0:16]
                    vb = row[16:32]
                    buf[r, 0:16] = va
                    buf[r, 16:32] = vb
                    return acc2 + va * va + vb * vb
                racc = lax.fori_loop(0, 128, _row0, racc)
            else:
                def _rowa(r, carry):
                    row = tmpb[r, :].astype(jnp.float32)
                    buf[r, 0:16] = buf[r, 0:16] + row[0:16]
                    buf[r, 16:32] = buf[r, 16:32] + row[16:32]
                    return carry
                lax.fori_loop(0, 128, _rowa, 0)
        return racc

    racc = load_set(uid, ue)
    racc = racc + load_set(iid, pe)
    racc = racc + load_set(nid, ne)

    def _prod(r, carry):
        ue[r, 0:16] = ue[r, 0:16] * (pe[r, 0:16] - ne[r, 0:16]) * 0.0625
        ue[r, 16:32] = ue[r, 16:32] * (pe[r, 16:32] - ne[r, 16:32]) * 0.0625
        return carry
    lax.fori_loop(0, 128, _prod, 0)

    regv[...] = racc
    pltpu.sync_copy(ue, m_out.at[pl.ds(b0, 128)])
    pltpu.sync_copy(regv, reg_out.at[pl.ds(pl.multiple_of(wid * 16, 16), 16)])


_score = functools.partial(
    pl.kernel,
    out_type=(jax.ShapeDtypeStruct((B, D), jnp.float32),
              jax.ShapeDtypeStruct((NC * NS * 16,), jnp.float32)),
    mesh=_mesh,
    scratch_types=[
        pltpu.VMEM((128,), jnp.int32),
        pltpu.VMEM((128, D), jnp.float32),
        pltpu.VMEM((128, D), jnp.float32),
        pltpu.VMEM((128, D), jnp.float32),
        pltpu.VMEM((128, D), jnp.bfloat16),
        pltpu.VMEM((16,), jnp.float32),
        pltpu.SemaphoreType.DMA,
    ],
    compiler_params=pltpu.CompilerParams(use_tc_tiling_on_sc=False),
)(_score_body)


def _loss_body(m_ref, r_ref, o_ref):
    d = jnp.sum(m_ref[...], axis=1)
    sg = 1.0 / (1.0 + jnp.exp(-d))
    bpr = -jnp.mean(jnp.log(sg))
    reg = jnp.sum(r_ref[...]) * (0.5 / B)
    o_ref[...] = jnp.full((8, 128), bpr + LMBD_C * reg, jnp.float32)


def kernel(user_emb, item_emb, edge_weight, edge_index, user_id, item_id, neg_item_id):
    all0 = jnp.concatenate([user_emb, item_emb], axis=0)
    e0b = all0.astype(jnp.bfloat16)
    pad = E_PAD - E
    src1 = jnp.pad(edge_index[0], (0, pad))
    dst1 = jnp.pad(edge_index[1], (0, pad))
    ew1 = jnp.pad(edge_weight, (0, pad))

    psrc, pdst, pew = _part(src1, dst1, ew1)
    psrc2 = psrc.reshape(-1, 128)
    pdst2 = pdst.reshape(-1, 128)
    pew2 = pew.reshape(-1, 128)
    e1 = _layer(psrc2, pdst2, pew2, e0b)
    e2 = _layer(psrc2, pdst2, pew2, e1)
    e3 = _layer(psrc2, pdst2, pew2, e2)

    mvec, regp = _score(e0b, e1, e2, e3,
                        user_id, item_id + U, neg_item_id + U)

    out = pl.pallas_call(
        _loss_body,
        out_shape=jax.ShapeDtypeStruct((8, 128), jnp.float32),
    )(mvec, regp.reshape(4, 128))
    return out[0, 0]


# speedup vs baseline: 1.0002x; 1.0002x over previous
import functools

import jax
import jax.numpy as jnp
from jax import lax
from jax.experimental import pallas as pl
from jax.experimental.pallas import tpu as pltpu
from jax.experimental.pallas import tpu_sc as plsc

U = 50000
NN = 100000
D = 32
E = 1600000
B = 4096
LMBD_C = 1e-4
HALF = 50000
ACC_ROWS = 50176
E_PAD = 1638400
P_C = 28672
SUP = 2048
NSUP = 2 * P_C // SUP
CPS = SUP // 128
NBUF = 4
NSC = 2
NGRP = CPS // NBUF
NC = 2
NS = 16
_mesh = plsc.VectorSubcoreMesh(
    core_axis_name="c", subcore_axis_name="s", num_cores=NC, num_subcores=NS)


def _layer_body(src_hbm, dst_hbm, ew_hbm, tab_hbm, out_hbm,
                st_src0, st_src1, st_dst0, st_dst1, st_ew0, st_ew1,
                sidx0, sidx1, sidx2, sidx3, didx0, didx1, didx2, didx3,
                rows0, rows1, rows2, rows3, wrows0, wrows1,
                acc, isem, gsem, ssem):
    st_src = (st_src0, st_src1)
    st_dst = (st_dst0, st_dst1)
    st_ew = (st_ew0, st_ew1)
    sidx = (sidx0, sidx1, sidx2, sidx3)
    didx = (didx0, didx1, didx2, didx3)
    rows = (rows0, rows1, rows2, rows3)
    wrows = (wrows0, wrows1)

    c = lax.axis_index("c")
    s = lax.axis_index("s")
    lane = lax.iota(jnp.int32, 16)
    zv = jnp.zeros((16,), jnp.float32)

    def _zb(i, carry):
        wrows0[i, 0:16] = zv
        wrows0[i, 16:32] = zv
        return carry
    lax.fori_loop(0, 128, _zb, 0)

    def _za(i, carry):
        off = pl.multiple_of(s * 3072 + i * 128, 128)
        pltpu.sync_copy(wrows0, acc.at[pl.ds(off, 128)])
        return carry
    lax.fori_loop(0, 24, _za, 0)

    @pl.when(s < 8)
    def _zrem():
        off = pl.multiple_of(49152 + s * 128, 128)
        pltpu.sync_copy(wrows0, acc.at[pl.ds(off, 128)])
    plsc.subcore_barrier()

    lo = c * HALF
    rbase = (c * 32 + 2 * s) * (P_C // 128)

    def _stage_start(u, par):
        row_base = pl.multiple_of(rbase + u * CPS, 8)
        pltpu.async_copy(src_hbm.at[pl.ds(row_base, CPS)], st_src[par], isem)
        pltpu.async_copy(dst_hbm.at[pl.ds(row_base, CPS)], st_dst[par], isem)
        pltpu.async_copy(ew_hbm.at[pl.ds(row_base, CPS)], st_ew[par], isem)

    def _stage_wait(u, par):
        row_base = pl.multiple_of(rbase + u * CPS, 8)
        pltpu.make_async_copy(src_hbm.at[pl.ds(row_base, CPS)], st_src[par], isem).wait()
        pltpu.make_async_copy(dst_hbm.at[pl.ds(row_base, CPS)], st_dst[par], isem).wait()
        pltpu.make_async_copy(ew_hbm.at[pl.ds(row_base, CPS)], st_ew[par], isem).wait()

    def _run_super(u, par):
        _stage_wait(u, par)

        def _grp(g, carry):
            gg = u * NGRP + g

            @pl.when(gg > 0)
            def _wait_prev():
                pltpu.make_async_copy(wrows[0], acc.at[didx[2]], ssem.at[0]).wait()
                pltpu.make_async_copy(wrows[1], acc.at[didx[3]], ssem.at[1]).wait()

            gdesc = []
            for b in range(NBUF):
                k = g * NBUF + b
                for j in range(8):
                    slj = pl.ds(j * 16, 16)
                    sidx[b][slj] = st_src[par][k, slj]
                    dv = st_dst[par][k, slj] - lo
                    m = (dv >= 0) & (dv < HALF) & (st_ew[par][k, slj] != 0.0)
                    didx[b][slj] = jnp.where(m, dv, HALF + lane + j * 16)
                gdesc.append(pltpu.async_copy(tab_hbm.at[sidx[b]], rows[b], gsem.at[b]))
            for b in range(NBUF):
                sb = b & 1
                if b >= NSC:
                    pltpu.make_async_copy(wrows[sb], acc.at[didx[b - NSC]], ssem.at[sb]).wait()
                gdesc[b].wait()
                k = g * NBUF + b
                for j in range(8):
                    w16 = st_ew[par][k, pl.ds(j * 16, 16)]
                    for i in range(16):
                        r = j * 16 + i
                        ws = w16[i]
                        row = rows[b][r, :].astype(jnp.float32)
                        wrows[sb][r, 0:16] = row[0:16] * ws
                        wrows[sb][r, 16:32] = row[16:32] * ws
                pltpu.async_copy(wrows[sb], acc.at[didx[b]], ssem.at[sb], add=True)
            return carry
        lax.fori_loop(0, NGRP, _grp, 0)

    _stage_start(0, 0)

    def _pair(m, carry):
        for par in range(2):
            u = 2 * m + par

            @pl.when(u + 1 < NSUP)
            def _prefetch():
                _stage_start(u + 1, 1 - par)
            _run_super(u, par)
        return carry
    lax.fori_loop(0, NSUP // 2, _pair, 0)

    pltpu.make_async_copy(wrows[0], acc.at[didx[2]], ssem.at[0]).wait()
    pltpu.make_async_copy(wrows[1], acc.at[didx[3]], ssem.at[1]).wait()
    plsc.subcore_barrier()

    def _pack_store(off, n):
        pltpu.sync_copy(acc.at[pl.ds(off, n)], wrows0.at[pl.ds(0, n)])

        def _pk(r, carry):
            rows0[r, :] = wrows0[r, :].astype(jnp.bfloat16)
            return carry
        lax.fori_loop(0, n, _pk, 0)
        pltpu.sync_copy(rows0.at[pl.ds(0, n)],
                        out_hbm.at[pl.ds(pl.multiple_of(lo + off, 8), n)])

    nfull = jnp.where(s == 15, 15, 25)

    def _co(i, carry):
        _pack_store(pl.multiple_of(s * 3200 + i * 128, 128), 128)
        return carry
    lax.fori_loop(0, nfull, _co, 0)

    @pl.when(s == 15)
    def _tail():
        _pack_store(15 * 3200 + 15 * 128, 80)


_layer = functools.partial(
    pl.kernel,
    out_type=jax.ShapeDtypeStruct((NN, D), jnp.bfloat16),
    mesh=_mesh,
    scratch_types=(
        [pltpu.VMEM((CPS, 128), jnp.int32)] * 4
        + [pltpu.VMEM((CPS, 128), jnp.float32)] * 2
        + [pltpu.VMEM((128,), jnp.int32)] * 8
        + [pltpu.VMEM((128, D), jnp.bfloat16)] * 4
        + [pltpu.VMEM((128, D), jnp.float32)] * 2
        + [
            pltpu.VMEM_SHARED((ACC_ROWS, D), jnp.float32),
            pltpu.SemaphoreType.DMA,
            pltpu.SemaphoreType.DMA((NBUF,)),
            pltpu.SemaphoreType.DMA((NSC,)),
        ]
    ),
    compiler_params=pltpu.CompilerParams(use_tc_tiling_on_sc=False),
)(_layer_body)


def _part_body(src_hbm, dst_hbm, ew_hbm, osrc, odst, oew,
               st_src, st_dst, st_ew, cs0, cs1, cd0, cd1, cw0, cw1):
    c = lax.axis_index("c")
    s = lax.axis_index("s")
    wid = s * NC + c
    zi = jnp.zeros((16,), jnp.int32)
    zf = jnp.zeros((16,), jnp.float32)
    cb = ((cs0, cd0, cw0), (cs1, cd1, cw1))

    def _flush(h, w):
        bufs = cb[h]
        off = pl.multiple_of(
            (h * 32 + wid) * P_C + jnp.minimum(w, P_C - 2048), 8)
        pltpu.sync_copy(bufs[0].at[pl.ds(0, 2048)], osrc.at[pl.ds(off, 2048)])
        pltpu.sync_copy(bufs[1].at[pl.ds(0, 2048)], odst.at[pl.ds(off, 2048)])
        pltpu.sync_copy(bufs[2].at[pl.ds(0, 2048)], oew.at[pl.ds(off, 2048)])

    def _super(u, carry):
        off = pl.multiple_of(wid * 51200 + u * 2048, 8)
        pltpu.sync_copy(src_hbm.at[pl.ds(off, 2048)], st_src)
        pltpu.sync_copy(dst_hbm.at[pl.ds(off, 2048)], st_dst)
        pltpu.sync_copy(ew_hbm.at[pl.ds(off, 2048)], st_ew)

        def _gr(g, car2):
            cnt0, cnt1, w0, w1 = car2
            sl = pl.ds(g * 16, 16)
            sv = st_src[sl]
            dv = st_dst[sl]
            wv = st_ew[sl]
            nz = wv != 0.0
            m0 = (dv < HALF) & nz
            m1 = (dv >= HALF) & nz
            cnts = [cnt0, cnt1]
            for h, mh in ((0, m0), (1, m1)):
                bufs = cb[h]
                cnt = cnts[h]
                plsc.store_compressed(bufs[0].at[pl.ds(cnt, 16)], sv, mask=mh)
                plsc.store_compressed(bufs[1].at[pl.ds(cnt, 16)], dv, mask=mh)
                plsc.store_compressed(bufs[2].at[pl.ds(cnt, 16)], wv, mask=mh)
                cnts[h] = cnt + plsc.all_reduce_population_count(mh)[0]
            cnt0, cnt1 = cnts
            ws = [w0, w1]
            for h in (0, 1):
                cnt = cnts[h]
                full = cnt >= 2048
                w = ws[h]

                @pl.when(full)
                def _do_flush():
                    _flush(h, w)
                    bufs = cb[h]
                    for bi in range(3):
                        bufs[bi][pl.ds(0, 16)] = bufs[bi][pl.ds(2048, 16)]
                cnts[h] = jnp.where(full, cnt - 2048, cnt)
                ws[h] = jnp.where(full, w + 2048, w)
            return (cnts[0], cnts[1], ws[0], ws[1])
        return lax.fori_loop(0, 128, _gr, carry)

    z32 = jnp.int32(0)
    cnt0, cnt1, w0, w1 = lax.fori_loop(0, 25, _super, (z32, z32, z32, z32))

    for h, cnt, w in ((0, cnt0, w0), (1, cnt1, w1)):
        bufs = cb[h]

        def _zt(i, carry):
            start = cnt + i * 16
            bufs[0][pl.ds(start, 16)] = zi
            bufs[1][pl.ds(start, 16)] = zi
            bufs[2][pl.ds(start, 16)] = zf
            return carry
        lax.fori_loop(0, (2048 - cnt + 15) // 16, _zt, 0)
        _flush(h, w)
        w = w + 2048

        def _zb(i, carry):
            sl = pl.ds(i * 16, 16)
            bufs[0][sl] = zi
            bufs[1][sl] = zi
            bufs[2][sl] = zf
            return carry
        lax.fori_loop(0, 128, _zb, 0)
        ndz = jnp.maximum(0, (P_C - w) // 2048)

        def _zfl(i, carry):
            _flush(h, jnp.minimum(w + i * 2048, P_C - 2048))
            return carry
        lax.fori_loop(0, ndz, _zfl, 0)


_part = functools.partial(
    pl.kernel,
    out_type=(jax.ShapeDtypeStruct((2 * 32 * P_C,), jnp.int32),
              jax.ShapeDtypeStruct((2 * 32 * P_C,), jnp.int32),
              jax.ShapeDtypeStruct((2 * 32 * P_C,), jnp.float32)),
    mesh=_mesh,
    scratch_types=(
        [pltpu.VMEM((2048,), jnp.int32)] * 2
        + [pltpu.VMEM((2048,), jnp.float32)]
        + [pltpu.VMEM((2064,), jnp.int32)] * 4
        + [pltpu.VMEM((2064,), jnp.float32)] * 2
    ),
    compiler_params=pltpu.CompilerParams(
        use_tc_tiling_on_sc=False, needs_layout_passes=False),
)(_part_body)


def _score_body(e0, e1, e2, e3, uid, iid, nid, m_out, reg_out,
                idx_v, ue, pe, ne, tmpb, regv, sem):
    c = lax.axis_index("c")
    s = lax.axis_index("s")
    wid = s * NC + c
    b0 = pl.multiple_of(wid * 128, 128)

    def load_set(ids_hbm, buf):
        pltpu.sync_copy(ids_hbm.at[pl.ds(b0, 128)], idx_v)
        racc = jnp.zeros((16,), jnp.float32)
        for ti, t in enumerate((e0, e1, e2, e3)):
            pltpu.async_copy(t.at[idx_v], tmpb, sem).wait()
            if ti == 0:
                def _row0(r, acc2):
                    row = tmpb[r, :].astype(jnp.float32)
                    va = row[0:16]
                    vb = row[16:32]
                    buf[r, 0:16] = va
                    buf[r, 16:32] = vb
                    return acc2 + va * va + vb * vb
                racc = lax.fori_loop(0, 128, _row0, racc)
            else:
                def _rowa(r, carry):
                    row = tmpb[r, :].astype(jnp.float32)
                    buf[r, 0:16] = buf[r, 0:16] + row[0:16]
                    buf[r, 16:32] = buf[r, 16:32] + row[16:32]
                    return carry
                lax.fori_loop(0, 128, _rowa, 0)
        return racc

    racc = load_set(uid, ue)
    racc = racc + load_set(iid, pe)
    racc = racc + load_set(nid, ne)

    def _prod(r, carry):
        ue[r, 0:16] = ue[r, 0:16] * (pe[r, 0:16] - ne[r, 0:16]) * 0.0625
        ue[r, 16:32] = ue[r, 16:32] * (pe[r, 16:32] - ne[r, 16:32]) * 0.0625
        return carry
    lax.fori_loop(0, 128, _prod, 0)

    regv[...] = racc
    pltpu.sync_copy(ue, m_out.at[pl.ds(b0, 128)])
    pltpu.sync_copy(regv, reg_out.at[pl.ds(pl.multiple_of(wid * 16, 16), 16)])


_score = functools.partial(
    pl.kernel,
    out_type=(jax.ShapeDtypeStruct((B, D), jnp.float32),
              jax.ShapeDtypeStruct((NC * NS * 16,), jnp.float32)),
    mesh=_mesh,
    scratch_types=[
        pltpu.VMEM((128,), jnp.int32),
        pltpu.VMEM((128, D), jnp.float32),
        pltpu.VMEM((128, D), jnp.float32),
        pltpu.VMEM((128, D), jnp.float32),
        pltpu.VMEM((128, D), jnp.bfloat16),
        pltpu.VMEM((16,), jnp.float32),
        pltpu.SemaphoreType.DMA,
    ],
    compiler_params=pltpu.CompilerParams(use_tc_tiling_on_sc=False),
)(_score_body)


def _loss_body(m_ref, r_ref, o_ref):
    d = jnp.sum(m_ref[...], axis=1)
    sg = 1.0 / (1.0 + jnp.exp(-d))
    bpr = -jnp.mean(jnp.log(sg))
    reg = jnp.sum(r_ref[...]) * (0.5 / B)
    o_ref[...] = jnp.full((8, 128), bpr + LMBD_C * reg, jnp.float32)


def kernel(user_emb, item_emb, edge_weight, edge_index, user_id, item_id, neg_item_id):
    all0 = jnp.concatenate([user_emb, item_emb], axis=0)
    e0b = all0.astype(jnp.bfloat16)
    pad = E_PAD - E
    src1 = jnp.pad(edge_index[0], (0, pad))
    dst1 = jnp.pad(edge_index[1], (0, pad))
    ew1 = jnp.pad(edge_weight, (0, pad))

    psrc, pdst, pew = _part(src1, dst1, ew1)
    psrc2 = psrc.reshape(-1, 128)
    pdst2 = pdst.reshape(-1, 128)
    pew2 = pew.reshape(-1, 128)
    e1 = _layer(psrc2, pdst2, pew2, e0b)
    e2 = _layer(psrc2, pdst2, pew2, e1)
    e3 = _layer(psrc2, pdst2, pew2, e2)

    mvec, regp = _score(e0b, e1, e2, e3,
                        user_id, item_id + U, neg_item_id + U)

    out = pl.pallas_call(
        _loss_body,
        out_shape=jax.ShapeDtypeStruct((8, 128), jnp.float32),
    )(mvec, regp.reshape(4, 128))
    return out[0, 0]


# speedup vs baseline: 1.6397x; 1.6393x over previous
import functools

import jax
import jax.numpy as jnp
from jax import lax
from jax.experimental import pallas as pl
from jax.experimental.pallas import tpu as pltpu
from jax.experimental.pallas import tpu_sc as plsc

U = 50000
NN = 100000
D = 32
E = 1600000
B = 4096
LMBD_C = 1e-4
HALF = 50000
ACC_ROWS = 50176
EPT = 102400
E_PAD = 16 * EPT
SUP = 2048
NSUP = EPT // SUP
CPS = SUP // 128
NBUF = 4
NSC = 2
NGRP = CPS // NBUF
NC = 2
NS = 16
_mesh = plsc.VectorSubcoreMesh(
    core_axis_name="c", subcore_axis_name="s", num_cores=NC, num_subcores=NS)


def _layer_body(src_hbm, dst_hbm, ew_hbm, tab_hbm, out_hbm,
                st_src0, st_src1, st_dst0, st_dst1, st_ew0, st_ew1,
                sidx0, sidx1, sidx2, sidx3, didx0, didx1, didx2, didx3,
                rows0, rows1, rows2, rows3, wrows0, wrows1,
                acc, isem, gsem, ssem):
    st_src = (st_src0, st_src1)
    st_dst = (st_dst0, st_dst1)
    st_ew = (st_ew0, st_ew1)
    sidx = (sidx0, sidx1, sidx2, sidx3)
    didx = (didx0, didx1, didx2, didx3)
    rows = (rows0, rows1, rows2, rows3)
    wrows = (wrows0, wrows1)

    c = lax.axis_index("c")
    s = lax.axis_index("s")
    lane = lax.iota(jnp.int32, 16)
    zv = jnp.zeros((16,), jnp.float32)

    def _zb(i, carry):
        wrows0[i, 0:16] = zv
        wrows0[i, 16:32] = zv
        return carry
    lax.fori_loop(0, 128, _zb, 0)

    def _za(i, carry):
        off = pl.multiple_of(s * 3072 + i * 128, 128)
        pltpu.sync_copy(wrows0, acc.at[pl.ds(off, 128)])
        return carry
    lax.fori_loop(0, 24, _za, 0)

    @pl.when(s < 8)
    def _zrem():
        off = pl.multiple_of(49152 + s * 128, 128)
        pltpu.sync_copy(wrows0, acc.at[pl.ds(off, 128)])
    plsc.subcore_barrier()

    lo = c * HALF

    def _stage_start(u, par):
        row_base = pl.multiple_of((s * EPT + u * SUP) // 128, 8)
        pltpu.async_copy(src_hbm.at[pl.ds(row_base, CPS)], st_src[par], isem)
        pltpu.async_copy(dst_hbm.at[pl.ds(row_base, CPS)], st_dst[par], isem)
        pltpu.async_copy(ew_hbm.at[pl.ds(row_base, CPS)], st_ew[par], isem)

    def _stage_wait(u, par):
        row_base = pl.multiple_of((s * EPT + u * SUP) // 128, 8)
        pltpu.make_async_copy(src_hbm.at[pl.ds(row_base, CPS)], st_src[par], isem).wait()
        pltpu.make_async_copy(dst_hbm.at[pl.ds(row_base, CPS)], st_dst[par], isem).wait()
        pltpu.make_async_copy(ew_hbm.at[pl.ds(row_base, CPS)], st_ew[par], isem).wait()

    def _run_super(u, par):
        _stage_wait(u, par)

        def _grp(g, carry):
            gg = u * NGRP + g

            @pl.when(gg > 0)
            def _wait_prev():
                pltpu.make_async_copy(wrows[0], acc.at[didx[2]], ssem.at[0]).wait()
                pltpu.make_async_copy(wrows[1], acc.at[didx[3]], ssem.at[1]).wait()

            gdesc = []
            for b in range(NBUF):
                k = g * NBUF + b
                for j in range(8):
                    slj = pl.ds(j * 16, 16)
                    sidx[b][slj] = st_src[par][k, slj]
                    dv = st_dst[par][k, slj] - lo
                    m = (dv >= 0) & (dv < HALF)
                    didx[b][slj] = jnp.where(m, dv, HALF + lane + j * 16)
                gdesc.append(pltpu.async_copy(tab_hbm.at[sidx[b]], rows[b], gsem.at[b]))
            for b in range(NBUF):
                sb = b & 1
                if b >= NSC:
                    pltpu.make_async_copy(wrows[sb], acc.at[didx[b - NSC]], ssem.at[sb]).wait()
                gdesc[b].wait()
                k = g * NBUF + b
                for j in range(8):
                    w16 = st_ew[par][k, pl.ds(j * 16, 16)]
                    for i in range(16):
                        r = j * 16 + i
                        ws = w16[i]
                        row = rows[b][r, :].astype(jnp.float32)
                        wrows[sb][r, 0:16] = row[0:16] * ws
                        wrows[sb][r, 16:32] = row[16:32] * ws
                pltpu.async_copy(wrows[sb], acc.at[didx[b]], ssem.at[sb], add=True)
            return carry
        lax.fori_loop(0, NGRP, _grp, 0)

    _stage_start(0, 0)

    def _pair(m, carry):
        for par in range(2):
            u = 2 * m + par

            @pl.when(u + 1 < NSUP)
            def _prefetch():
                _stage_start(u + 1, 1 - par)
            _run_super(u, par)
        return carry
    lax.fori_loop(0, NSUP // 2, _pair, 0)

    pltpu.make_async_copy(wrows[0], acc.at[didx[2]], ssem.at[0]).wait()
    pltpu.make_async_copy(wrows[1], acc.at[didx[3]], ssem.at[1]).wait()
    plsc.subcore_barrier()

    def _pack_store(off, n):
        pltpu.sync_copy(acc.at[pl.ds(off, n)], wrows0.at[pl.ds(0, n)])

        def _pk(r, carry):
            rows0[r, :] = wrows0[r, :].astype(jnp.bfloat16)
            return carry
        lax.fori_loop(0, n, _pk, 0)
        pltpu.sync_copy(rows0.at[pl.ds(0, n)],
                        out_hbm.at[pl.ds(pl.multiple_of(lo + off, 8), n)])

    nfull = jnp.where(s == 15, 15, 25)

    def _co(i, carry):
        _pack_store(pl.multiple_of(s * 3200 + i * 128, 128), 128)
        return carry
    lax.fori_loop(0, nfull, _co, 0)

    @pl.when(s == 15)
    def _tail():
        _pack_store(15 * 3200 + 15 * 128, 80)


_layer = functools.partial(
    pl.kernel,
    out_type=jax.ShapeDtypeStruct((NN, D), jnp.bfloat16),
    mesh=_mesh,
    scratch_types=(
        [pltpu.VMEM((CPS, 128), jnp.int32)] * 4
        + [pltpu.VMEM((CPS, 128), jnp.float32)] * 2
        + [pltpu.VMEM((128,), jnp.int32)] * 8
        + [pltpu.VMEM((128, D), jnp.bfloat16)] * 4
        + [pltpu.VMEM((128, D), jnp.float32)] * 2
        + [
            pltpu.VMEM_SHARED((ACC_ROWS, D), jnp.float32),
            pltpu.SemaphoreType.DMA,
            pltpu.SemaphoreType.DMA((NBUF,)),
            pltpu.SemaphoreType.DMA((NSC,)),
        ]
    ),
    compiler_params=pltpu.CompilerParams(use_tc_tiling_on_sc=False),
)(_layer_body)


def _score_body(e0, e1, e2, e3, uid, iid, nid, m_out, reg_out,
                idx_v, ue, pe, ne, tmpb, regv, sem):
    c = lax.axis_index("c")
    s = lax.axis_index("s")
    wid = s * NC + c
    b0 = pl.multiple_of(wid * 128, 128)

    def load_set(ids_hbm, buf):
        pltpu.sync_copy(ids_hbm.at[pl.ds(b0, 128)], idx_v)
        racc = jnp.zeros((16,), jnp.float32)
        for ti, t in enumerate((e0, e1, e2, e3)):
            pltpu.async_copy(t.at[idx_v], tmpb, sem).wait()
            if ti == 0:
                def _row0(r, acc2):
                    row = tmpb[r, :].astype(jnp.float32)
                    va = row[0:16]
                    vb = row[16:32]
                    buf[r, 0:16] = va
                    buf[r, 16:32] = vb
                    return acc2 + va * va + vb * vb
                racc = lax.fori_loop(0, 128, _row0, racc)
            else:
                def _rowa(r, carry):
                    row = tmpb[r, :].astype(jnp.float32)
                    buf[r, 0:16] = buf[r, 0:16] + row[0:16]
                    buf[r, 16:32] = buf[r, 16:32] + row[16:32]
                    return carry
                lax.fori_loop(0, 128, _rowa, 0)
        return racc

    racc = load_set(uid, ue)
    racc = racc + load_set(iid, pe)
    racc = racc + load_set(nid, ne)

    def _prod(r, carry):
        ue[r, 0:16] = ue[r, 0:16] * (pe[r, 0:16] - ne[r, 0:16]) * 0.0625
        ue[r, 16:32] = ue[r, 16:32] * (pe[r, 16:32] - ne[r, 16:32]) * 0.0625
        return carry
    lax.fori_loop(0, 128, _prod, 0)

    regv[...] = racc
    pltpu.sync_copy(ue, m_out.at[pl.ds(b0, 128)])
    pltpu.sync_copy(regv, reg_out.at[pl.ds(pl.multiple_of(wid * 16, 16), 16)])


_score = functools.partial(
    pl.kernel,
    out_type=(jax.ShapeDtypeStruct((B, D), jnp.float32),
              jax.ShapeDtypeStruct((NC * NS * 16,), jnp.float32)),
    mesh=_mesh,
    scratch_types=[
        pltpu.VMEM((128,), jnp.int32),
        pltpu.VMEM((128, D), jnp.float32),
        pltpu.VMEM((128, D), jnp.float32),
        pltpu.VMEM((128, D), jnp.float32),
        pltpu.VMEM((128, D), jnp.bfloat16),
        pltpu.VMEM((16,), jnp.float32),
        pltpu.SemaphoreType.DMA,
    ],
    compiler_params=pltpu.CompilerParams(use_tc_tiling_on_sc=False),
)(_score_body)


def _loss_body(m_ref, r_ref, o_ref):
    d = jnp.sum(m_ref[...], axis=1)
    sg = 1.0 / (1.0 + jnp.exp(-d))
    bpr = -jnp.mean(jnp.log(sg))
    reg = jnp.sum(r_ref[...]) * (0.5 / B)
    o_ref[...] = jnp.full((8, 128), bpr + LMBD_C * reg, jnp.float32)


def kernel(user_emb, item_emb, edge_weight, edge_index, user_id, item_id, neg_item_id):
    all0 = jnp.concatenate([user_emb, item_emb], axis=0)
    e0b = all0.astype(jnp.bfloat16)
    pad = E_PAD - E
    src2 = jnp.pad(edge_index[0], (0, pad)).reshape(-1, 128)
    dst2 = jnp.pad(edge_index[1], (0, pad)).reshape(-1, 128)
    ew2 = jnp.pad(edge_weight, (0, pad)).reshape(-1, 128)

    e1 = _layer(src2, dst2, ew2, e0b)
    e2 = _layer(src2, dst2, ew2, e1)
    e3 = _layer(src2, dst2, ew2, e2)

    mvec, regp = _score(e0b, e1, e2, e3,
                        user_id, item_id + U, neg_item_id + U)

    out = pl.pallas_call(
        _loss_body,
        out_shape=jax.ShapeDtypeStruct((8, 128), jnp.float32),
    )(mvec, regp.reshape(4, 128))
    return out[0, 0]
